# baseline (speedup 1.0000x reference)
import functools

import jax
import jax.numpy as jnp
from jax import lax
from jax.experimental import pallas as pl
from jax.experimental.pallas import tpu as pltpu
from jax.experimental.pallas import tpu_sc as plsc

_E = 320000
_D = 128
_NC = 2
_NS = 16
_NW = _NC * _NS
_PER_W = _E // _NW
_K = 80
_NCHUNK = _PER_W // _K
_L = 16


def _dot_chunk(srows, drows, obuf):
    lane = jnp.arange(_L, dtype=jnp.int32)

    def gbody(g, carry):
        def ebody(i, vec):
            e = g * _L + i
            acc0 = jnp.zeros((_L,), jnp.float32)
            acc1 = jnp.zeros((_L,), jnp.float32)
            for j in range(_D // (2 * _L)):
                acc0 = acc0 + (srows[e, pl.ds(2 * j * _L, _L)] *
                               drows[e, pl.ds(2 * j * _L, _L)])
                acc1 = acc1 + (srows[e, pl.ds((2 * j + 1) * _L, _L)] *
                               drows[e, pl.ds((2 * j + 1) * _L, _L)])
            return jnp.where(lane == i, jnp.sum(acc0 + acc1), vec)

        vec = lax.fori_loop(0, _L, ebody, jnp.zeros((_L,), jnp.float32),
                            unroll=4)
        obuf[pl.ds(pl.multiple_of(g * _L, _L), _L)] = vec
        return carry

    lax.fori_loop(0, _K // _L, gbody, 0)


_NBUF = 5

@functools.partial(
    pl.kernel,
    out_type=jax.ShapeDtypeStruct((_E,), jnp.float32),
    mesh=plsc.VectorSubcoreMesh(core_axis_name="c", subcore_axis_name="s"),
    compiler_params=pltpu.CompilerParams(needs_layout_passes=False),
    scratch_types=[
        pltpu.VMEM((_NBUF, _K), jnp.int32),
        pltpu.VMEM((_NBUF, _K), jnp.int32),
        pltpu.VMEM((_NBUF, _K, _D), jnp.float32),
        pltpu.VMEM((_NBUF, _K, _D), jnp.float32),
        pltpu.VMEM((_NBUF, _K), jnp.float32),
    ] + [pltpu.SemaphoreType.DMA] * (3 * _NBUF),
)
def _score_kernel(x_hbm, src_hbm, dst_hbm, out_hbm,
                  sidx, didx, srows, drows, obuf, *sems):
    wid = lax.axis_index("s") * _NC + lax.axis_index("c")
    sem_i = sems[0:_NBUF]
    sem_g = sems[_NBUF:2 * _NBUF]
    sem_o = sems[2 * _NBUF:3 * _NBUF]

    def issue_idx(c, b):
        base = pl.multiple_of(wid * _PER_W + c * _K, _K)
        pltpu.async_copy(src_hbm.at[pl.ds(base, _K)], sidx.at[b], sem_i[b])
        pltpu.async_copy(dst_hbm.at[pl.ds(base, _K)], didx.at[b], sem_i[b])

    def wait_idx(b):
        pltpu.make_async_copy(src_hbm.at[pl.ds(0, _K)], sidx.at[b],
                              sem_i[b]).wait()
        pltpu.make_async_copy(dst_hbm.at[pl.ds(0, _K)], didx.at[b],
                              sem_i[b]).wait()

    def issue_gather(b):
        pltpu.async_copy(x_hbm.at[sidx.at[b]], srows.at[b], sem_g[b])
        pltpu.async_copy(x_hbm.at[didx.at[b]], drows.at[b], sem_g[b])

    def wait_gather(b):
        pltpu.make_async_copy(x_hbm.at[pl.ds(0, _K)], srows.at[b],
                              sem_g[b]).wait()
        pltpu.make_async_copy(x_hbm.at[pl.ds(0, _K)], drows.at[b],
                              sem_g[b]).wait()

    def wait_out(b):
        pltpu.make_async_copy(obuf.at[b], out_hbm.at[pl.ds(0, _K)],
                              sem_o[b]).wait()

    def step(c, b):
        gb = (b + _NBUF - 1) % _NBUF

        @pl.when(c + _NBUF - 1 < _NCHUNK)
        def _():
            wait_idx(gb)
            issue_gather(gb)

        wait_gather(b)

        @pl.when(c + _NBUF < _NCHUNK)
        def _():
            issue_idx(c + _NBUF, b)

        @pl.when(c >= _NBUF)
        def _():
            wait_out(b)

        _dot_chunk(srows.at[b], drows.at[b], obuf.at[b])
        base = pl.multiple_of(wid * _PER_W + c * _K, _K)
        pltpu.async_copy(obuf.at[b], out_hbm.at[pl.ds(base, _K)], sem_o[b])

    for c in range(_NBUF):
        issue_idx(c, c)
    for c in range(_NBUF - 1):
        wait_idx(c)
        issue_gather(c)

    def quad(j, carry):
        for b in range(_NBUF):
            step(j * _NBUF + b, b)
        return carry

    lax.fori_loop(0, _NCHUNK // _NBUF, quad, 0)
    for b in range(_NBUF):
        wait_out(b)


def kernel(x, edge_index):
    src = edge_index[0].astype(jnp.int32)
    dst = edge_index[1].astype(jnp.int32)
    out = _score_kernel(x, src, dst)
    return out.reshape(_E, 1)


# speedup vs baseline: 1.0556x; 1.0556x over previous
import functools

import jax
import jax.numpy as jnp
from jax import lax
from jax.experimental import pallas as pl
from jax.experimental.pallas import tpu as pltpu
from jax.experimental.pallas import tpu_sc as plsc

_E = 320000
_D = 128
_NC = 2
_NS = 16
_NW = _NC * _NS
_PER_W = _E // _NW
_K = 40
_NCHUNK = _PER_W // _K
_L = 16


def _dot_chunk(srows, drows, obuf):
    lane = jnp.arange(_L, dtype=jnp.int32)

    def gbody(g, carry):
        def ebody(i, vec):
            e = g * _L + i
            acc0 = jnp.zeros((_L,), jnp.float32)
            acc1 = jnp.zeros((_L,), jnp.float32)
            for j in range(_D // (2 * _L)):
                acc0 = acc0 + (srows[e, pl.ds(2 * j * _L, _L)] *
                               drows[e, pl.ds(2 * j * _L, _L)])
                acc1 = acc1 + (srows[e, pl.ds((2 * j + 1) * _L, _L)] *
                               drows[e, pl.ds((2 * j + 1) * _L, _L)])
            return jnp.where(lane == i, jnp.sum(acc0 + acc1), vec)

        vec = lax.fori_loop(0, _L, ebody, jnp.zeros((_L,), jnp.float32),
                            unroll=4)
        obuf[pl.ds(pl.multiple_of(g * _L, _L), _L)] = vec
        return carry

    lax.fori_loop(0, _K // _L, gbody, 0)


_NBUF = 4

@functools.partial(
    pl.kernel,
    out_type=jax.ShapeDtypeStruct((_E,), jnp.float32),
    mesh=plsc.VectorSubcoreMesh(core_axis_name="c", subcore_axis_name="s"),
    compiler_params=pltpu.CompilerParams(needs_layout_passes=False),
    scratch_types=[
        pltpu.VMEM((_NBUF, _K), jnp.int32),
        pltpu.VMEM((_NBUF, _K), jnp.int32),
        pltpu.VMEM((_NBUF, _K, _D), jnp.float32),
        pltpu.VMEM((_NBUF, _K, _D), jnp.float32),
        pltpu.VMEM((_NBUF, _K), jnp.float32),
        pltpu.VMEM_SHARED((10000, _D), jnp.float32),
    ] + [pltpu.SemaphoreType.DMA] * (3 * _NBUF + 1),
)
def _score_kernel(x_hbm, src_hbm, dst_hbm, out_hbm,
                  sidx, didx, srows, drows, obuf, xsh, *sems):
    wid = lax.axis_index("s") * _NC + lax.axis_index("c")
    sem_i = sems[0:_NBUF]
    sem_g = sems[_NBUF:2 * _NBUF]
    sem_o = sems[2 * _NBUF:3 * _NBUF]
    sem_x = sems[3 * _NBUF]

    def issue_idx(c, b):
        base = pl.multiple_of(wid * _PER_W + c * _K, _K)
        pltpu.async_copy(src_hbm.at[pl.ds(base, _K)], sidx.at[b], sem_i[b])
        pltpu.async_copy(dst_hbm.at[pl.ds(base, _K)], didx.at[b], sem_i[b])

    def wait_idx(b):
        pltpu.make_async_copy(src_hbm.at[pl.ds(0, _K)], sidx.at[b],
                              sem_i[b]).wait()
        pltpu.make_async_copy(dst_hbm.at[pl.ds(0, _K)], didx.at[b],
                              sem_i[b]).wait()

    def issue_gather(b):
        pltpu.async_copy(xsh.at[sidx.at[b]], srows.at[b], sem_g[b])
        pltpu.async_copy(xsh.at[didx.at[b]], drows.at[b], sem_g[b])

    def wait_gather(b):
        pltpu.make_async_copy(x_hbm.at[pl.ds(0, _K)], srows.at[b],
                              sem_g[b]).wait()
        pltpu.make_async_copy(x_hbm.at[pl.ds(0, _K)], drows.at[b],
                              sem_g[b]).wait()

    def wait_out(b):
        pltpu.make_async_copy(obuf.at[b], out_hbm.at[pl.ds(0, _K)],
                              sem_o[b]).wait()

    def step(c, b):
        gb = (b + _NBUF - 1) % _NBUF

        @pl.when(c + _NBUF - 1 < _NCHUNK)
        def _():
            wait_idx(gb)
            issue_gather(gb)

        wait_gather(b)

        @pl.when(c + _NBUF < _NCHUNK)
        def _():
            issue_idx(c + _NBUF, b)

        @pl.when(c >= _NBUF)
        def _():
            wait_out(b)

        _dot_chunk(srows.at[b], drows.at[b], obuf.at[b])
        base = pl.multiple_of(wid * _PER_W + c * _K, _K)
        pltpu.async_copy(obuf.at[b], out_hbm.at[pl.ds(base, _K)], sem_o[b])

    sid = lax.axis_index("s")
    row0 = pl.multiple_of(sid * 624, 8)
    xcp = pltpu.async_copy(x_hbm.at[pl.ds(row0, 624)],
                           xsh.at[pl.ds(row0, 624)], sem_x)

    @pl.when(sid == 0)
    def _():
        pltpu.sync_copy(x_hbm.at[pl.ds(9984, 16)], xsh.at[pl.ds(9984, 16)])

    for c in range(_NBUF):
        issue_idx(c, c)
    xcp.wait()
    plsc.subcore_barrier()
    for c in range(_NBUF - 1):
        wait_idx(c)
        issue_gather(c)

    def quad(j, carry):
        for b in range(_NBUF):
            step(j * _NBUF + b, b)
        return carry

    lax.fori_loop(0, _NCHUNK // _NBUF, quad, 0)
    for r in range(_NCHUNK % _NBUF):
        c = _NCHUNK - (_NCHUNK % _NBUF) + r
        step(c, c % _NBUF)
    for b in range(_NBUF):
        wait_out(b)


def kernel(x, edge_index):
    src = edge_index[0].astype(jnp.int32)
    dst = edge_index[1].astype(jnp.int32)
    out = _score_kernel(x, src, dst)
    return out.reshape(_E, 1)
